# baseline (device time: 37633 ns/iter reference)
import jax
import jax.numpy as jnp
from jax import lax
from jax.experimental import pallas as pl
from jax.experimental.pallas import tpu as pltpu

N_DEV = 4
B = 2
S_PER = 128
D = 512
H = 8
DH = 64
SCALE = 0.125


def kernel(x, Wq, Wo, Wk, Wv):
    def body(x_ref, wq_ref, wo_ref, wk_ref, wv_ref, out_ref,
             x0, xl, xr, xd, q_ref, k_ref, v_ref,
             p0, p1, p2, p3, a1, a2, a3,
             ag_send, ag_recv, rs_send, rs_recv):
        my = lax.axis_index("i")
        left = lax.rem(my + N_DEV - 1, N_DEV)
        right = lax.rem(my + 1, N_DEV)
        diag = lax.rem(my + 2, N_DEV)

        barrier_sem = pltpu.get_barrier_semaphore()
        for nbr in (left, right, diag):
            pl.semaphore_signal(
                barrier_sem, inc=1,
                device_id=(nbr,), device_id_type=pl.DeviceIdType.MESH,
            )
        pl.semaphore_wait(barrier_sem, 3)

        def send(src, dst, dev, sem_i, send_sems, recv_sems):
            rdma = pltpu.make_async_remote_copy(
                src_ref=src, dst_ref=dst,
                send_sem=send_sems.at[sem_i],
                recv_sem=recv_sems.at[sem_i],
                device_id=(dev,),
                device_id_type=pl.DeviceIdType.MESH,
            )
            rdma.start()
            return rdma

        x0[...] = x_ref[...].astype(jnp.bfloat16)
        dl = send(x0, xr, left, 0, ag_send, ag_recv)
        dr = send(x0, xl, right, 1, ag_send, ag_recv)
        dd = send(x0, xd, diag, 2, ag_send, ag_recv)

        xslots = (x0, xl, xd, xr)
        prots = (p0, p1, p2, p3)

        w_qkv = jnp.concatenate(
            [wq_ref[...] * SCALE, wk_ref[...], wv_ref[...]], axis=1
        ).astype(jnp.bfloat16)
        wo = wo_ref[...].astype(jnp.bfloat16)

        def qkv(c):
            rows = slice(c * S_PER, (c + 1) * S_PER)
            for b in range(B):
                qkvb = jnp.dot(
                    xslots[c][b], w_qkv, preferred_element_type=jnp.float32
                )
                q_ref[b, rows, :] = qkvb[:, 0 * D:1 * D]
                k_ref[b, rows, :] = qkvb[:, 1 * D:2 * D]
                v_ref[b, rows, :] = qkvb[:, 2 * D:3 * D]

        qkv(0)
        dr.wait()
        qkv(1)
        dl.wait()
        qkv(3)
        dd.wait()
        qkv(2)

        def partial_chunk(c):
            rows = slice(c * S_PER, (c + 1) * S_PER)
            for b in range(B):
                heads = []
                for hh in range(H):
                    cols = slice(hh * DH, (hh + 1) * DH)
                    qh = q_ref[b, rows, cols]
                    kh = k_ref[b, :, cols]
                    vh = v_ref[b, :, cols]
                    s = lax.dot_general(
                        qh, kh, (((1,), (1,)), ((), ())),
                        preferred_element_type=jnp.float32,
                    )
                    p = jnp.exp(s)
                    l = jnp.sum(p, axis=-1, keepdims=True)
                    pv = jnp.dot(p, vh, preferred_element_type=jnp.float32)
                    heads.append(pv * jnp.reciprocal(l))
                o = jnp.concatenate(heads, axis=1)
                prots[c][b] = jnp.dot(
                    o, wo, preferred_element_type=jnp.float32
                ).astype(jnp.bfloat16)

        partial_chunk(2)
        sd = send(p2, a3, diag, 2, rs_send, rs_recv)
        partial_chunk(1)
        sl = send(p1, a1, left, 0, rs_send, rs_recv)
        partial_chunk(3)
        sr = send(p3, a2, right, 1, rs_send, rs_recv)
        partial_chunk(0)
        sl.wait()
        sr.wait()
        sd.wait()
        out_ref[...] = (
            p0[...].astype(jnp.float32) + a1[...].astype(jnp.float32)
        ) + (
            a2[...].astype(jnp.float32) + a3[...].astype(jnp.float32)
        )

    chunk = pltpu.VMEM((B, S_PER, D), jnp.bfloat16)
    return pl.pallas_call(
        body,
        out_shape=jax.ShapeDtypeStruct((B, S_PER, D), jnp.float32),
        in_specs=[pl.BlockSpec(memory_space=pltpu.VMEM)] * 5,
        out_specs=pl.BlockSpec(memory_space=pltpu.VMEM),
        scratch_shapes=[
            chunk, chunk, chunk, chunk,
            pltpu.VMEM((B, N_DEV * S_PER, D), jnp.float32),
            pltpu.VMEM((B, N_DEV * S_PER, D), jnp.float32),
            pltpu.VMEM((B, N_DEV * S_PER, D), jnp.float32),
            chunk, chunk, chunk, chunk,
            chunk, chunk, chunk,
            pltpu.SemaphoreType.DMA((3,)),
            pltpu.SemaphoreType.DMA((3,)),
            pltpu.SemaphoreType.DMA((3,)),
            pltpu.SemaphoreType.DMA((3,)),
        ],
        compiler_params=pltpu.CompilerParams(collective_id=0),
    )(x, Wq, Wo, Wk, Wv)


# device time: 28328 ns/iter; 1.3285x vs baseline; 1.3285x over previous
import jax
import jax.numpy as jnp
from jax import lax
from jax.experimental import pallas as pl
from jax.experimental.pallas import tpu as pltpu

N_DEV = 4
B = 2
S_PER = 128
D = 512
H = 8
DH = 64
SCALE = 0.125


def kernel(x, Wq, Wo, Wk, Wv):
    def body(x_ref, wq_ref, wo_ref, wk_ref, wv_ref, out_ref,
             x0, xl, xr, xd, q_ref, k_ref, v_ref, o_ref,
             p0, p1, p2, p3, a1, a2, a3,
             ag_send, ag_recv, rs_send, rs_recv):
        my = lax.axis_index("i")
        left = lax.rem(my + N_DEV - 1, N_DEV)
        right = lax.rem(my + 1, N_DEV)
        diag = lax.rem(my + 2, N_DEV)

        barrier_sem = pltpu.get_barrier_semaphore()
        for nbr in (left, right, diag):
            pl.semaphore_signal(
                barrier_sem, inc=1,
                device_id=(nbr,), device_id_type=pl.DeviceIdType.MESH,
            )
        pl.semaphore_wait(barrier_sem, 3)

        def send(src, dst, dev, sem_i, send_sems, recv_sems):
            rdma = pltpu.make_async_remote_copy(
                src_ref=src, dst_ref=dst,
                send_sem=send_sems.at[sem_i],
                recv_sem=recv_sems.at[sem_i],
                device_id=(dev,),
                device_id_type=pl.DeviceIdType.MESH,
            )
            rdma.start()
            return rdma

        x0[...] = x_ref[...].astype(jnp.bfloat16)
        dl0 = send(x0.at[0], xr.at[0], left, 0, ag_send, ag_recv)
        dr0 = send(x0.at[0], xl.at[0], right, 2, ag_send, ag_recv)
        dd0 = send(x0.at[0], xd.at[0], diag, 4, ag_send, ag_recv)
        dl1 = send(x0.at[1], xr.at[1], left, 1, ag_send, ag_recv)
        dr1 = send(x0.at[1], xl.at[1], right, 3, ag_send, ag_recv)
        dd1 = send(x0.at[1], xd.at[1], diag, 5, ag_send, ag_recv)

        xslots = (x0, xl, xd, xr)
        prots = (p0, p1, p2, p3)

        w_qkv = jnp.concatenate(
            [wq_ref[...] * SCALE, wk_ref[...], wv_ref[...]], axis=1
        ).astype(jnp.bfloat16)
        wo = wo_ref[...].astype(jnp.bfloat16)

        def qkv(c, b):
            rows = slice(c * S_PER, (c + 1) * S_PER)
            qkvb = jnp.dot(
                xslots[c][b], w_qkv, preferred_element_type=jnp.float32
            ).astype(jnp.bfloat16)
            q_ref[b, rows, :] = qkvb[:, 0 * D:1 * D]
            k_ref[b, rows, :] = qkvb[:, 1 * D:2 * D]
            v_ref[b, rows, :] = qkvb[:, 2 * D:3 * D]

        qkv(0, 0)
        qkv(0, 1)
        dr0.wait()
        qkv(1, 0)
        dl0.wait()
        qkv(3, 0)
        dd0.wait()
        qkv(2, 0)

        def attn_batch(b):
            for hh in range(H):
                cols = slice(hh * DH, (hh + 1) * DH)
                qh = q_ref[b, :, cols]
                kh = k_ref[b, :, cols]
                vh = v_ref[b, :, cols]
                s = lax.dot_general(
                    qh, kh, (((1,), (1,)), ((), ())),
                    preferred_element_type=jnp.float32,
                )
                p = jnp.exp(s.astype(jnp.bfloat16))
                vh_ext = jnp.concatenate(
                    [vh, jnp.ones((N_DEV * S_PER, 1), jnp.bfloat16)], axis=1)
                pv = jnp.dot(p, vh_ext, preferred_element_type=jnp.float32)
                o_ref[b, :, cols] = (
                    pv[:, :DH] * jnp.reciprocal(pv[:, DH:])
                ).astype(jnp.bfloat16)

        def partial_chunk(c, b):
            rows = slice(c * S_PER, (c + 1) * S_PER)
            prots[c][b] = jnp.dot(
                o_ref[b, rows, :], wo,
                preferred_element_type=jnp.float32,
            ).astype(jnp.bfloat16)

        def send_batch(b):
            partial_chunk(2, b)
            sd = send(p2.at[b], a3.at[b], diag, 0 + b, rs_send, rs_recv)
            partial_chunk(1, b)
            sl = send(p1.at[b], a1.at[b], left, 2 + b, rs_send, rs_recv)
            partial_chunk(3, b)
            sr = send(p3.at[b], a2.at[b], right, 4 + b, rs_send, rs_recv)
            return sd, sl, sr

        attn_batch(0)
        rs0 = send_batch(0)
        dr1.wait()
        qkv(1, 1)
        dl1.wait()
        qkv(3, 1)
        dd1.wait()
        qkv(2, 1)
        attn_batch(1)
        rs1 = send_batch(1)
        partial_chunk(0, 0)
        for r in rs0:
            r.wait()
        out_ref[0] = (
            p0[0].astype(jnp.float32) + a1[0].astype(jnp.float32)
        ) + (
            a2[0].astype(jnp.float32) + a3[0].astype(jnp.float32)
        )
        partial_chunk(0, 1)
        for r in rs1:
            r.wait()
        out_ref[1] = (
            p0[1].astype(jnp.float32) + a1[1].astype(jnp.float32)
        ) + (
            a2[1].astype(jnp.float32) + a3[1].astype(jnp.float32)
        )

    chunk = pltpu.VMEM((B, S_PER, D), jnp.bfloat16)
    return pl.pallas_call(
        body,
        out_shape=jax.ShapeDtypeStruct((B, S_PER, D), jnp.float32),
        in_specs=[pl.BlockSpec(memory_space=pltpu.VMEM)] * 5,
        out_specs=pl.BlockSpec(memory_space=pltpu.VMEM),
        scratch_shapes=[
            chunk, chunk, chunk, chunk,
            pltpu.VMEM((B, N_DEV * S_PER, D), jnp.bfloat16),
            pltpu.VMEM((B, N_DEV * S_PER, D), jnp.bfloat16),
            pltpu.VMEM((B, N_DEV * S_PER, D), jnp.bfloat16),
            pltpu.VMEM((B, N_DEV * S_PER, D), jnp.bfloat16),
            chunk, chunk, chunk, chunk,
            chunk, chunk, chunk,
            pltpu.SemaphoreType.DMA((6,)),
            pltpu.SemaphoreType.DMA((6,)),
            pltpu.SemaphoreType.DMA((6,)),
            pltpu.SemaphoreType.DMA((6,)),
        ],
        compiler_params=pltpu.CompilerParams(collective_id=0),
    )(x, Wq, Wo, Wk, Wv)
